# baseline (speedup 1.0000x reference)
import functools

import jax
import jax.numpy as jnp
from jax import lax
from jax.experimental import pallas as pl
from jax.experimental.pallas import tpu as pltpu

_BLK_K = 2048


def _mm_kernel(a_ref, b_ref, o_ref, *, nsteps, tail):
    k = pl.program_id(0)

    @pl.when(k == 0)
    def _init():
        o_ref[...] = jnp.zeros_like(o_ref)

    @pl.when(k < nsteps - 1)
    def _body():
        o_ref[...] += jnp.dot(
            a_ref[...], b_ref[...], preferred_element_type=jnp.float32
        )

    @pl.when(k == nsteps - 1)
    def _tail():
        a = a_ref[...]
        b = b_ref[...]
        col = lax.broadcasted_iota(jnp.int32, a.shape, 1)
        a = jnp.where(col < tail, a, 0.0)
        row = lax.broadcasted_iota(jnp.int32, b.shape, 0)
        b = jnp.where(row < tail, b, 0.0)
        o_ref[...] += jnp.dot(a, b, preferred_element_type=jnp.float32)


def kernel(attention_weights, content_matrix):
    m, k_dim = attention_weights.shape
    _, n = content_matrix.shape
    nsteps = pl.cdiv(k_dim, _BLK_K)
    tail = k_dim - (nsteps - 1) * _BLK_K
    body = functools.partial(_mm_kernel, nsteps=nsteps, tail=tail)
    return pl.pallas_call(
        body,
        grid=(nsteps,),
        in_specs=[
            pl.BlockSpec((m, _BLK_K), lambda k: (0, k)),
            pl.BlockSpec((_BLK_K, n), lambda k: (k, 0)),
        ],
        out_specs=pl.BlockSpec((m, n), lambda k: (0, 0)),
        out_shape=jax.ShapeDtypeStruct((m, n), jnp.float32),
        compiler_params=pltpu.CompilerParams(
            dimension_semantics=("arbitrary",)
        ),
    )(attention_weights, content_matrix)
